# baseline (speedup 1.0000x reference)
import functools

import jax
import jax.numpy as jnp
from jax import lax
from jax.experimental import pallas as pl
from jax.experimental.pallas import tpu as pltpu
from jax.experimental.pallas import tpu_sc as plsc

N = 10000
E = 320000
D = 128
CW = 16

NC, NS = 2, 16
NW = NC * NS
CHUNK = 64
EPW = E // NW
PCHUNK = 8
NPASS = 20
NCHUNK = PCHUNK * NPASS
EPW_PAD = NCHUNK * CHUNK
XROWS = 10240
NPHASE = 4
PROWS = 2560
AROWS = 2688
ZPT = AROWS // NS
OPT = PROWS // NS
DUMMY = PROWS
DEPTH = 2


def _sc_body(x_hbm, src_hbm, dst_hbm, sum_out, cnt_out,
             x_sp, acc, acc_cnt,
             is0, is1, id0, id1, dstbuf, rb0, rb1, ones_v, zcnt,
             gs0, gs1, ss0, ss1, rsem):
    c = lax.axis_index("c")
    s = lax.axis_index("s")
    w = c * NS + s
    idx_src = (is0, is1)
    idx_dst = (id0, id1)
    rowbuf = (rb0, rb1)
    gsem = (gs0, gs1)
    ssem = (ss0, ss1)

    zrow = jnp.zeros((16,), jnp.float32)

    @pl.loop(0, CHUNK)
    def _fill(i):
        ones_v[i, :] = jnp.ones((16,), jnp.float32)
        zcnt[i, :] = zrow

    for k in range(XROWS // NS // CHUNK):
        r = s * (XROWS // NS) + k * CHUNK
        pltpu.sync_copy(x_hbm.at[pl.ds(r, CHUNK)], rb0)
        pltpu.sync_copy(rb0, x_sp.at[pl.ds(r, CHUNK)])

    @pl.loop(0, NPHASE)
    def _phase(p):
            base = p * PROWS
            @pl.loop(0, CHUNK)
            def _zfill(i):
                for k in range(D // 16):
                    rb0[i, pl.ds(k * 16, 16)] = zrow
            for k in range(2):
                pltpu.sync_copy(rb0, acc.at[pl.ds(s * ZPT + k * CHUNK, CHUNK)])
                pltpu.sync_copy(zcnt, acc_cnt.at[pl.ds(s * ZPT + k * CHUNK, CHUNK)])
            pltpu.sync_copy(rb0.at[pl.ds(0, ZPT - 2 * CHUNK)],
                            acc.at[pl.ds(s * ZPT + 2 * CHUNK, ZPT - 2 * CHUNK)])
            pltpu.sync_copy(zcnt.at[pl.ds(0, ZPT - 2 * CHUNK)],
                            acc_cnt.at[pl.ds(s * ZPT + 2 * CHUNK, ZPT - 2 * CHUNK)])
            plsc.subcore_barrier()

            r_desc = [
                pltpu.async_copy(src_hbm.at[w, pl.ds(0, PCHUNK)], is0, rsem),
                pltpu.async_copy(dst_hbm.at[w, pl.ds(0, PCHUNK)], id0, rsem),
            ]
            g_desc = [None] * DEPTH
            s_desc = [None] * DEPTH
            c_desc = [None] * DEPTH
            pending = [None] * NCHUNK
            t = 0
            for q in range(NPASS):
                hs, hd = idx_src[q % 2], idx_dst[q % 2]
                for d_ in r_desc:
                    d_.wait()
                if q + 1 < NPASS:
                    nhs, nhd = idx_src[(q + 1) % 2], idx_dst[(q + 1) % 2]
                    r_desc = [
                        pltpu.async_copy(
                            src_hbm.at[w, pl.ds((q + 1) * PCHUNK, PCHUNK)],
                            nhs, rsem),
                        pltpu.async_copy(
                            dst_hbm.at[w, pl.ds((q + 1) * PCHUNK, PCHUNK)],
                            nhd, rsem),
                    ]
                for j in range(PCHUNK):
                    b = t % DEPTH
                    if s_desc[b] is not None:
                        s_desc[b].wait()
                        c_desc[b].wait()
                    g_desc[b] = pltpu.async_copy(
                        x_sp.at[hs.at[j]], rowbuf[b], gsem[b])
                    for k in range(CHUNK // 16):
                        dv = hd[j, pl.ds(k * 16, 16)] - base
                        ok = (dv >= 0) & (dv < PROWS)
                        dstbuf[b, pl.ds(k * 16, 16)] = jnp.where(ok, dv, DUMMY)
                    pending[t] = b
                    tp = t - 1
                    if tp >= 0:
                        pb = pending[tp]
                        g_desc[pb].wait()
                        s_desc[pb] = pltpu.async_copy(
                            rowbuf[pb], acc.at[dstbuf.at[pb]], ssem[pb], add=True)
                        c_desc[pb] = pltpu.async_copy(
                            ones_v, acc_cnt.at[dstbuf.at[pb]], ssem[pb], add=True)
                    t += 1
            pb = pending[NCHUNK - 1]
            g_desc[pb].wait()
            s_desc[pb] = pltpu.async_copy(
                rowbuf[pb], acc.at[dstbuf.at[pb]], ssem[pb], add=True)
            c_desc[pb] = pltpu.async_copy(
                ones_v, acc_cnt.at[dstbuf.at[pb]], ssem[pb], add=True)
            for b in range(DEPTH):
                if s_desc[b] is not None:
                    s_desc[b].wait()
                    c_desc[b].wait()
            plsc.subcore_barrier()

            pltpu.sync_copy(acc.at[pl.ds(s * OPT, OPT)],
                            sum_out.at[c, p, pl.ds(s * OPT, OPT)])
            pltpu.sync_copy(acc_cnt.at[pl.ds(s * OPT, OPT)],
                            cnt_out.at[c, p, pl.ds(s * OPT, OPT)])
            plsc.subcore_barrier()


_sc_scatter = functools.partial(
    pl.kernel,
    out_type=[
        jax.ShapeDtypeStruct((NC, NPHASE, PROWS, D), jnp.float32),
        jax.ShapeDtypeStruct((NC, NPHASE, PROWS, CW), jnp.float32),
    ],
    mesh=plsc.VectorSubcoreMesh(core_axis_name="c", subcore_axis_name="s"),
    scratch_types=(
        [pltpu.VMEM_SHARED((XROWS, D), jnp.float32),
         pltpu.VMEM_SHARED((AROWS, D), jnp.float32),
         pltpu.VMEM_SHARED((AROWS, CW), jnp.float32)]
        + [pltpu.VMEM((PCHUNK, CHUNK), jnp.int32)] * 4
        + [pltpu.VMEM((DEPTH, CHUNK), jnp.int32)]
        + [pltpu.VMEM((CHUNK, D), jnp.float32)] * DEPTH
        + [pltpu.VMEM((CHUNK, CW), jnp.float32)] * 2
        + [pltpu.SemaphoreType.DMA] * (2 * DEPTH + 1)
    ),
    compiler_params=pltpu.CompilerParams(use_tc_tiling_on_sc=False),
)(_sc_body)


def _tc_body(x_ref, s0_ref, s1_ref, c0_ref, c1_ref, wl_ref, wr_ref, b_ref,
             o_ref):
    cnt = c0_ref[:, 0:1] + c1_ref[:, 0:1]
    agg = (s0_ref[...] + s1_ref[...]) / jnp.maximum(cnt, 1.0)
    dn = (((1,), (1,)), ((), ()))
    o_ref[...] = (
        lax.dot_general(agg, wl_ref[...], dn, preferred_element_type=jnp.float32)
        + lax.dot_general(x_ref[...], wr_ref[...], dn,
                          preferred_element_type=jnp.float32)
        + b_ref[...])


def _tc_dense(x, s0, s1, c0, c1, W_l, b_l, W_r):
    blk = 1000
    grid = N // blk
    row_spec = pl.BlockSpec((blk, D), lambda i: (i, 0))
    cnt_spec = pl.BlockSpec((blk, CW), lambda i: (i, 0))
    full = pl.BlockSpec((D, D), lambda i: (0, 0))
    bias = pl.BlockSpec((1, D), lambda i: (0, 0))
    return pl.pallas_call(
        _tc_body,
        grid=(grid,),
        in_specs=[row_spec, row_spec, row_spec, cnt_spec, cnt_spec,
                  full, full, bias],
        out_specs=row_spec,
        out_shape=jax.ShapeDtypeStruct((N, D), jnp.float32),
    )(x, s0, s1, c0, c1, W_l, W_r, b_l.reshape(1, D))


def kernel(x, edge_index, node_type, edge_type, W_l, b_l, W_r):
    x_pad = jnp.concatenate([x, jnp.zeros((XROWS - N, D), jnp.float32)])
    src = edge_index[0].reshape(NW, EPW)
    dst = edge_index[1].reshape(NW, EPW)
    pad = EPW_PAD - EPW
    src_p = jnp.concatenate(
        [src, jnp.zeros((NW, pad), jnp.int32)], axis=1).reshape(NW, NCHUNK, CHUNK)
    dst_p = jnp.concatenate(
        [dst, jnp.full((NW, pad), -1, jnp.int32)], axis=1).reshape(NW, NCHUNK, CHUNK)
    sums, cnts = _sc_scatter(x_pad, src_p, dst_p)
    sums = sums.reshape(NC, NPHASE * PROWS, D)
    cnts = cnts.reshape(NC, NPHASE * PROWS, CW)
    return _tc_dense(x, sums[0, :N], sums[1, :N], cnts[0, :N], cnts[1, :N],
                     W_l, b_l, W_r)


# speedup vs baseline: 1.6032x; 1.6032x over previous
import functools

import jax
import jax.numpy as jnp
from jax import lax
from jax.experimental import pallas as pl
from jax.experimental.pallas import tpu as pltpu
from jax.experimental.pallas import tpu_sc as plsc

N = 10000
E = 320000
D = 128
CW = 16

NC, NS = 2, 16
NW = NC * NS
CHUNK = 64
EPW = E // NW
PCHUNK = 8
NPASS = 20
NCHUNK = PCHUNK * NPASS
EPW_PAD = NCHUNK * CHUNK
XROWS = 10240
NPHASE = 4
PROWS = 2560
AROWS = 2688
ZPT = AROWS // NS
OPT = PROWS // NS
DUMMY = PROWS
DEPTH = 2


def _sc_body(x_hbm, src_hbm, dst_hbm, sum_out, cnt_out,
             x_sp, acc, acc_cnt,
             is0, is1, id0, id1, dstbuf, rb0, rb1, ones_v, zcnt,
             gs0, gs1, ss0, ss1, rsem):
    c = lax.axis_index("c")
    s = lax.axis_index("s")
    w = c * NS + s
    idx_src = (is0, is1)
    idx_dst = (id0, id1)
    rowbuf = (rb0, rb1)
    gsem = (gs0, gs1)
    ssem = (ss0, ss1)

    zrow = jnp.zeros((16,), jnp.float32)

    @pl.loop(0, CHUNK)
    def _fill(i):
        ones_v[i, :] = jnp.ones((16,), jnp.float32)
        zcnt[i, :] = zrow

    for k in range(XROWS // NS // CHUNK):
        r = s * (XROWS // NS) + k * CHUNK
        pltpu.sync_copy(x_hbm.at[pl.ds(r, CHUNK)], rb0)
        pltpu.sync_copy(rb0, x_sp.at[pl.ds(r, CHUNK)])

    @pl.loop(0, NPHASE)
    def _phase(p):
            base = p * PROWS
            @pl.loop(0, CHUNK)
            def _zfill(i):
                for k in range(D // 16):
                    rb0[i, pl.ds(k * 16, 16)] = zrow
            for k in range(2):
                pltpu.sync_copy(rb0, acc.at[pl.ds(s * ZPT + k * CHUNK, CHUNK)])
                pltpu.sync_copy(zcnt, acc_cnt.at[pl.ds(s * ZPT + k * CHUNK, CHUNK)])
            pltpu.sync_copy(rb0.at[pl.ds(0, ZPT - 2 * CHUNK)],
                            acc.at[pl.ds(s * ZPT + 2 * CHUNK, ZPT - 2 * CHUNK)])
            pltpu.sync_copy(zcnt.at[pl.ds(0, ZPT - 2 * CHUNK)],
                            acc_cnt.at[pl.ds(s * ZPT + 2 * CHUNK, ZPT - 2 * CHUNK)])
            plsc.subcore_barrier()

            r_desc = [
                pltpu.async_copy(src_hbm.at[w, pl.ds(0, PCHUNK)], is0, rsem),
                pltpu.async_copy(dst_hbm.at[w, pl.ds(0, PCHUNK)], id0, rsem),
            ]
            g_desc = [None] * DEPTH
            s_desc = [None] * DEPTH
            c_desc = [None] * DEPTH
            pending = [None] * NCHUNK
            t = 0
            for q in range(NPASS):
                hs, hd = idx_src[q % 2], idx_dst[q % 2]
                for d_ in r_desc:
                    d_.wait()
                if q + 1 < NPASS:
                    nhs, nhd = idx_src[(q + 1) % 2], idx_dst[(q + 1) % 2]
                    r_desc = [
                        pltpu.async_copy(
                            src_hbm.at[w, pl.ds((q + 1) * PCHUNK, PCHUNK)],
                            nhs, rsem),
                        pltpu.async_copy(
                            dst_hbm.at[w, pl.ds((q + 1) * PCHUNK, PCHUNK)],
                            nhd, rsem),
                    ]
                for j in range(PCHUNK):
                    b = t % DEPTH
                    if s_desc[b] is not None:
                        s_desc[b].wait()
                        c_desc[b].wait()
                    g_desc[b] = pltpu.async_copy(
                        x_sp.at[hs.at[j]], rowbuf[b], gsem[b])
                    for k in range(CHUNK // 16):
                        du = hd[j, pl.ds(k * 16, 16)]
                        dv = du - base
                        ok = (dv >= 0) & (dv < PROWS)
                        dstbuf[b, pl.ds(k * 16, 16)] = jnp.where(
                            ok, dv, DUMMY + (du & 127))
                    pending[t] = b
                    tp = t - 1
                    if tp >= 0:
                        pb = pending[tp]
                        g_desc[pb].wait()
                        s_desc[pb] = pltpu.async_copy(
                            rowbuf[pb], acc.at[dstbuf.at[pb]], ssem[pb], add=True)
                        c_desc[pb] = pltpu.async_copy(
                            ones_v, acc_cnt.at[dstbuf.at[pb]], ssem[pb], add=True)
                    t += 1
            pb = pending[NCHUNK - 1]
            g_desc[pb].wait()
            s_desc[pb] = pltpu.async_copy(
                rowbuf[pb], acc.at[dstbuf.at[pb]], ssem[pb], add=True)
            c_desc[pb] = pltpu.async_copy(
                ones_v, acc_cnt.at[dstbuf.at[pb]], ssem[pb], add=True)
            for b in range(DEPTH):
                if s_desc[b] is not None:
                    s_desc[b].wait()
                    c_desc[b].wait()
            plsc.subcore_barrier()

            pltpu.sync_copy(acc.at[pl.ds(s * OPT, OPT)],
                            sum_out.at[c, p, pl.ds(s * OPT, OPT)])
            pltpu.sync_copy(acc_cnt.at[pl.ds(s * OPT, OPT)],
                            cnt_out.at[c, p, pl.ds(s * OPT, OPT)])
            plsc.subcore_barrier()


_sc_scatter = functools.partial(
    pl.kernel,
    out_type=[
        jax.ShapeDtypeStruct((NC, NPHASE, PROWS, D), jnp.float32),
        jax.ShapeDtypeStruct((NC, NPHASE, PROWS, CW), jnp.float32),
    ],
    mesh=plsc.VectorSubcoreMesh(core_axis_name="c", subcore_axis_name="s"),
    scratch_types=(
        [pltpu.VMEM_SHARED((XROWS, D), jnp.float32),
         pltpu.VMEM_SHARED((AROWS, D), jnp.float32),
         pltpu.VMEM_SHARED((AROWS, CW), jnp.float32)]
        + [pltpu.VMEM((PCHUNK, CHUNK), jnp.int32)] * 4
        + [pltpu.VMEM((DEPTH, CHUNK), jnp.int32)]
        + [pltpu.VMEM((CHUNK, D), jnp.float32)] * DEPTH
        + [pltpu.VMEM((CHUNK, CW), jnp.float32)] * 2
        + [pltpu.SemaphoreType.DMA] * (2 * DEPTH + 1)
    ),
    compiler_params=pltpu.CompilerParams(use_tc_tiling_on_sc=False),
)(_sc_body)


def _tc_body(x_ref, s0_ref, s1_ref, c0_ref, c1_ref, wl_ref, wr_ref, b_ref,
             o_ref):
    cnt = c0_ref[:, 0:1] + c1_ref[:, 0:1]
    agg = (s0_ref[...] + s1_ref[...]) / jnp.maximum(cnt, 1.0)
    dn = (((1,), (1,)), ((), ()))
    o_ref[...] = (
        lax.dot_general(agg, wl_ref[...], dn, preferred_element_type=jnp.float32)
        + lax.dot_general(x_ref[...], wr_ref[...], dn,
                          preferred_element_type=jnp.float32)
        + b_ref[...])


def _tc_dense(x, s0, s1, c0, c1, W_l, b_l, W_r):
    blk = 1000
    grid = N // blk
    row_spec = pl.BlockSpec((blk, D), lambda i: (i, 0))
    cnt_spec = pl.BlockSpec((blk, CW), lambda i: (i, 0))
    full = pl.BlockSpec((D, D), lambda i: (0, 0))
    bias = pl.BlockSpec((1, D), lambda i: (0, 0))
    return pl.pallas_call(
        _tc_body,
        grid=(grid,),
        in_specs=[row_spec, row_spec, row_spec, cnt_spec, cnt_spec,
                  full, full, bias],
        out_specs=row_spec,
        out_shape=jax.ShapeDtypeStruct((N, D), jnp.float32),
    )(x, s0, s1, c0, c1, W_l, W_r, b_l.reshape(1, D))


def kernel(x, edge_index, node_type, edge_type, W_l, b_l, W_r):
    x_pad = jnp.concatenate([x, jnp.zeros((XROWS - N, D), jnp.float32)])
    src = edge_index[0].reshape(NW, EPW)
    dst = edge_index[1].reshape(NW, EPW)
    pad = EPW_PAD - EPW
    src_p = jnp.concatenate(
        [src, jnp.zeros((NW, pad), jnp.int32)], axis=1).reshape(NW, NCHUNK, CHUNK)
    dst_p = jnp.concatenate(
        [dst, jnp.full((NW, pad), -1, jnp.int32)], axis=1).reshape(NW, NCHUNK, CHUNK)
    sums, cnts = _sc_scatter(x_pad, src_p, dst_p)
    sums = sums.reshape(NC, NPHASE * PROWS, D)
    cnts = cnts.reshape(NC, NPHASE * PROWS, CW)
    return _tc_dense(x, sums[0, :N], sums[1, :N], cnts[0, :N], cnts[1, :N],
                     W_l, b_l, W_r)


# speedup vs baseline: 2.2137x; 1.3809x over previous
import functools

import jax
import jax.numpy as jnp
from jax import lax
from jax.experimental import pallas as pl
from jax.experimental.pallas import tpu as pltpu
from jax.experimental.pallas import tpu_sc as plsc

N = 10000
E = 320000
D = 128
DA = 144

NC, NS = 2, 16
NW = NC * NS
CHUNK = 64
EPW = E // NW
PCHUNK = 8
NPASS = 20
NCHUNK = PCHUNK * NPASS
EPW_PAD = NCHUNK * CHUNK
ROWS_ACC = 10112
RPT = ROWS_ACC // NS
DUMMY = N
DEPTH = 4


def _sc_body(x_hbm, src_hbm, dst_hbm, sum_out,
             acc, idx_src0, idx_src1, idx_dst0, idx_dst1,
             rb0, rb1, rb2, rb3,
             gs0, gs1, gs2, gs3, ss0, ss1, ss2, ss3):
    c = lax.axis_index("c")
    s = lax.axis_index("s")
    w = c * NS + s
    r0 = s * RPT
    idx_src = (idx_src0, idx_src1)
    idx_dst = (idx_dst0, idx_dst1)
    rowbuf = (rb0, rb1, rb2, rb3)
    gsem = (gs0, gs1, gs2, gs3)
    ssem = (ss0, ss1, ss2, ss3)

    zrow = jnp.zeros((16,), jnp.float32)

    @pl.loop(0, CHUNK)
    def _fill(i):
        for k in range(DA // 16):
            rb0[i, pl.ds(k * 16, 16)] = zrow

    for k in range(9):
        pltpu.sync_copy(rb0, acc.at[pl.ds(r0 + k * CHUNK, CHUNK)])
    pltpu.sync_copy(rb0.at[pl.ds(0, RPT - 9 * CHUNK)],
                    acc.at[pl.ds(r0 + 9 * CHUNK, RPT - 9 * CHUNK)])
    plsc.subcore_barrier()

    g_desc = [None] * DEPTH
    s_desc = [None] * DEPTH
    pending = [None] * NCHUNK
    t = 0
    for p in range(NPASS):
        hs, hd = idx_src[p % 2], idx_dst[p % 2]
        pltpu.sync_copy(src_hbm.at[w, pl.ds(p * PCHUNK, PCHUNK)], hs)
        pltpu.sync_copy(dst_hbm.at[w, pl.ds(p * PCHUNK, PCHUNK)], hd)
        for j in range(PCHUNK):
            b = t % DEPTH
            if s_desc[b] is not None:
                s_desc[b].wait()
            g_desc[b] = pltpu.async_copy(
                x_hbm.at[hs.at[j]], rowbuf[b], gsem[b])
            pending[t] = (b, hd.at[j])
            tp = t - 2
            if tp >= 0:
                pb, prow = pending[tp]
                g_desc[pb].wait()
                s_desc[pb] = pltpu.async_copy(
                    rowbuf[pb], acc.at[prow], ssem[pb], add=True)
            t += 1
    for tp in (NCHUNK - 2, NCHUNK - 1):
        pb, prow = pending[tp]
        g_desc[pb].wait()
        s_desc[pb] = pltpu.async_copy(
            rowbuf[pb], acc.at[prow], ssem[pb], add=True)
    for b in range(DEPTH):
        if s_desc[b] is not None:
            s_desc[b].wait()

    plsc.subcore_barrier()
    pltpu.sync_copy(acc.at[pl.ds(r0, RPT)], sum_out.at[c, pl.ds(r0, RPT)])


_sc_scatter = functools.partial(
    pl.kernel,
    out_type=[
        jax.ShapeDtypeStruct((NC, ROWS_ACC, DA), jnp.float32),
    ],
    mesh=plsc.VectorSubcoreMesh(core_axis_name="c", subcore_axis_name="s"),
    scratch_types=(
        [pltpu.VMEM_SHARED((ROWS_ACC, DA), jnp.float32)]
        + [pltpu.VMEM((PCHUNK, CHUNK), jnp.int32)] * 4
        + [pltpu.VMEM((CHUNK, DA), jnp.float32)] * DEPTH
        + [pltpu.SemaphoreType.DMA] * (2 * DEPTH)
    ),
    compiler_params=pltpu.CompilerParams(use_tc_tiling_on_sc=False),
)(_sc_body)


def _tc_body(x_ref, s0_ref, s1_ref, wl_ref, wr_ref, b_ref, o_ref):
    cnt = s0_ref[:, D:D + 1] + s1_ref[:, D:D + 1]
    agg = (s0_ref[:, :D] + s1_ref[:, :D]) / jnp.maximum(cnt, 1.0)
    dn = (((1,), (1,)), ((), ()))
    o_ref[...] = (
        lax.dot_general(agg, wl_ref[...], dn, preferred_element_type=jnp.float32)
        + lax.dot_general(x_ref[...], wr_ref[...], dn,
                          preferred_element_type=jnp.float32)
        + b_ref[...])


def _tc_dense(x, s0, s1, W_l, b_l, W_r):
    blk = 1000
    grid = N // blk
    row_spec = pl.BlockSpec((blk, D), lambda i: (i, 0))
    aug_spec = pl.BlockSpec((blk, DA), lambda i: (i, 0))
    full = pl.BlockSpec((D, D), lambda i: (0, 0))
    bias = pl.BlockSpec((1, D), lambda i: (0, 0))
    return pl.pallas_call(
        _tc_body,
        grid=(grid,),
        in_specs=[row_spec, aug_spec, aug_spec, full, full, bias],
        out_specs=row_spec,
        out_shape=jax.ShapeDtypeStruct((N, D), jnp.float32),
    )(x, s0, s1, W_l, W_r, b_l.reshape(1, D))


def kernel(x, edge_index, node_type, edge_type, W_l, b_l, W_r):
    x_aug = jnp.concatenate([x, jnp.ones((N, DA - D), jnp.float32)], axis=1)
    src = edge_index[0].reshape(NW, EPW)
    dst = edge_index[1].reshape(NW, EPW)
    pad = EPW_PAD - EPW
    src_p = jnp.concatenate(
        [src, jnp.zeros((NW, pad), jnp.int32)], axis=1).reshape(NW, NCHUNK, CHUNK)
    dst_p = jnp.concatenate(
        [dst, jnp.full((NW, pad), DUMMY, jnp.int32)], axis=1).reshape(NW, NCHUNK, CHUNK)
    (sums,) = _sc_scatter(x_aug, src_p, dst_p)
    return _tc_dense(x, sums[0, :N], sums[1, :N], W_l, b_l, W_r)


# speedup vs baseline: 2.3280x; 1.0516x over previous
import functools

import jax
import jax.numpy as jnp
from jax import lax
from jax.experimental import pallas as pl
from jax.experimental.pallas import tpu as pltpu
from jax.experimental.pallas import tpu_sc as plsc

N = 10000
E = 320000
D = 128
DA = 144

NC, NS = 2, 16
NW = NC * NS
CHUNK = 64
EPW = E // NW
PCHUNK = 8
NPASS = 20
NCHUNK = PCHUNK * NPASS
EPW_PAD = NCHUNK * CHUNK
ROWS_ACC = 10112
RPT = ROWS_ACC // NS
DUMMY = N
DEPTH = 4


def _sc_body(x_hbm, src_hbm, dst_hbm, sum_out,
             acc, idx_src0, idx_src1, idx_dst0, idx_dst1,
             rb0, rb1, rb2, rb3,
             gs0, gs1, gs2, gs3, ss0, ss1, ss2, ss3, rsem):
    c = lax.axis_index("c")
    s = lax.axis_index("s")
    w = c * NS + s
    r0 = s * RPT
    idx_src = (idx_src0, idx_src1)
    idx_dst = (idx_dst0, idx_dst1)
    rowbuf = (rb0, rb1, rb2, rb3)
    gsem = (gs0, gs1, gs2, gs3)
    ssem = (ss0, ss1, ss2, ss3)

    zrow = jnp.zeros((16,), jnp.float32)

    @pl.loop(0, CHUNK)
    def _fill(i):
        for k in range(DA // 16):
            rb0[i, pl.ds(k * 16, 16)] = zrow

    for k in range(9):
        pltpu.sync_copy(rb0, acc.at[pl.ds(r0 + k * CHUNK, CHUNK)])
    pltpu.sync_copy(rb0.at[pl.ds(0, RPT - 9 * CHUNK)],
                    acc.at[pl.ds(r0 + 9 * CHUNK, RPT - 9 * CHUNK)])
    plsc.subcore_barrier()

    r_desc = [
        pltpu.async_copy(src_hbm.at[w, pl.ds(0, PCHUNK)], idx_src[0], rsem),
        pltpu.async_copy(dst_hbm.at[w, pl.ds(0, PCHUNK)], idx_dst[0], rsem),
    ]
    g_desc = [None] * DEPTH
    s_desc = [None] * DEPTH
    pending = [None] * NCHUNK
    t = 0
    for p in range(NPASS):
        hs, hd = idx_src[p % 2], idx_dst[p % 2]
        for d_ in r_desc:
            d_.wait()
        for j in range(PCHUNK):
            if j == 4 and p + 1 < NPASS:
                r_desc = [
                    pltpu.async_copy(
                        src_hbm.at[w, pl.ds((p + 1) * PCHUNK, PCHUNK)],
                        idx_src[(p + 1) % 2], rsem),
                    pltpu.async_copy(
                        dst_hbm.at[w, pl.ds((p + 1) * PCHUNK, PCHUNK)],
                        idx_dst[(p + 1) % 2], rsem),
                ]
            b = t % DEPTH
            if s_desc[b] is not None:
                s_desc[b].wait()
            g_desc[b] = pltpu.async_copy(
                x_hbm.at[hs.at[j]], rowbuf[b], gsem[b])
            pending[t] = (b, hd.at[j])
            tp = t - 2
            if tp >= 0:
                pb, prow = pending[tp]
                g_desc[pb].wait()
                s_desc[pb] = pltpu.async_copy(
                    rowbuf[pb], acc.at[prow], ssem[pb], add=True)
            t += 1
    for tp in (NCHUNK - 2, NCHUNK - 1):
        pb, prow = pending[tp]
        g_desc[pb].wait()
        s_desc[pb] = pltpu.async_copy(
            rowbuf[pb], acc.at[prow], ssem[pb], add=True)
    for b in range(DEPTH):
        if s_desc[b] is not None:
            s_desc[b].wait()

    plsc.subcore_barrier()
    pltpu.sync_copy(acc.at[pl.ds(r0, RPT)], sum_out.at[c, pl.ds(r0, RPT)])


_sc_scatter = functools.partial(
    pl.kernel,
    out_type=[
        jax.ShapeDtypeStruct((NC, ROWS_ACC, DA), jnp.float32),
    ],
    mesh=plsc.VectorSubcoreMesh(core_axis_name="c", subcore_axis_name="s"),
    scratch_types=(
        [pltpu.VMEM_SHARED((ROWS_ACC, DA), jnp.float32)]
        + [pltpu.VMEM((PCHUNK, CHUNK), jnp.int32)] * 4
        + [pltpu.VMEM((CHUNK, DA), jnp.float32)] * DEPTH
        + [pltpu.SemaphoreType.DMA] * (2 * DEPTH + 1)
    ),
    compiler_params=pltpu.CompilerParams(use_tc_tiling_on_sc=False),
)(_sc_body)


def _tc_body(x_ref, s0_ref, s1_ref, wl_ref, wr_ref, b_ref, o_ref):
    cnt = s0_ref[:, D:D + 1] + s1_ref[:, D:D + 1]
    agg = (s0_ref[:, :D] + s1_ref[:, :D]) / jnp.maximum(cnt, 1.0)
    dn = (((1,), (1,)), ((), ()))
    o_ref[...] = (
        lax.dot_general(agg, wl_ref[...], dn, preferred_element_type=jnp.float32)
        + lax.dot_general(x_ref[...], wr_ref[...], dn,
                          preferred_element_type=jnp.float32)
        + b_ref[...])


def _tc_dense(x, s0, s1, W_l, b_l, W_r):
    blk = 1000
    grid = N // blk
    row_spec = pl.BlockSpec((blk, D), lambda i: (i, 0))
    aug_spec = pl.BlockSpec((blk, DA), lambda i: (i, 0))
    full = pl.BlockSpec((D, D), lambda i: (0, 0))
    bias = pl.BlockSpec((1, D), lambda i: (0, 0))
    return pl.pallas_call(
        _tc_body,
        grid=(grid,),
        in_specs=[row_spec, aug_spec, aug_spec, full, full, bias],
        out_specs=row_spec,
        out_shape=jax.ShapeDtypeStruct((N, D), jnp.float32),
    )(x, s0, s1, W_l, W_r, b_l.reshape(1, D))


def kernel(x, edge_index, node_type, edge_type, W_l, b_l, W_r):
    x_aug = jnp.concatenate([x, jnp.ones((N, DA - D), jnp.float32)], axis=1)
    src = edge_index[0].reshape(NW, EPW)
    dst = edge_index[1].reshape(NW, EPW)
    pad = EPW_PAD - EPW
    src_p = jnp.concatenate(
        [src, jnp.zeros((NW, pad), jnp.int32)], axis=1).reshape(NW, NCHUNK, CHUNK)
    dst_p = jnp.concatenate(
        [dst, jnp.full((NW, pad), DUMMY, jnp.int32)], axis=1).reshape(NW, NCHUNK, CHUNK)
    (sums,) = _sc_scatter(x_aug, src_p, dst_p)
    return _tc_dense(x, sums[0, :N], sums[1, :N], W_l, b_l, W_r)
